# Initial kernel scaffold; baseline (speedup 1.0000x reference)
import jax
import jax.numpy as jnp
from jax.experimental import pallas as pl


def kernel(feat, edge_index, W_self0, W_neigh0, b0, W_self1, W_neigh1, b1):
    raise NotImplementedError("write your pallas kernel here")


# speedup vs baseline: 6.6346x; 6.6346x over previous
import functools

import jax
import jax.numpy as jnp
from jax import lax
from jax.experimental import pallas as pl
from jax.experimental.pallas import tpu as pltpu
from jax.experimental.pallas import tpu_sc as plsc

NC = 2
NS = 16
LANES = 16
CH = 128
DEGW = 16


def _sc_aggregate(table, src, dst, with_deg):
    n, d = table.shape
    e = src.shape[0]
    nch = e // CH
    nw = NC * NS
    rpt = (n // NS) // 8 * 8
    tail = n - NS * rpt
    zrows = 16
    assert rpt % zrows == 0 and tail % zrows == 0

    mesh = plsc.VectorSubcoreMesh(
        core_axis_name="c", subcore_axis_name="s",
        num_cores=NC, num_subcores=NS)

    out_type = [jax.ShapeDtypeStruct((NC, n, d), jnp.float32)]
    scratch = [
        pltpu.VMEM((CH,), jnp.int32),
        pltpu.VMEM((CH,), jnp.int32),
        pltpu.VMEM((CH, d), jnp.float32),
        pltpu.VMEM((zrows, d), jnp.float32),
        pltpu.VMEM_SHARED((n, d), jnp.float32),
        pltpu.SemaphoreType.DMA,
    ]
    if with_deg:
        out_type.append(jax.ShapeDtypeStruct((NC, n, DEGW), jnp.float32))
        scratch += [
            pltpu.VMEM((CH, DEGW), jnp.float32),
            pltpu.VMEM((zrows, DEGW), jnp.float32),
            pltpu.VMEM_SHARED((n, DEGW), jnp.float32),
        ]

    def body(table_h, src_h, dst_h, acc_out, *rest):
        if with_deg:
            (deg_out, sidx, didx, rows, zb, acc_sh, gsem,
             ones, zbd, deg_sh) = rest
        else:
            sidx, didx, rows, zb, acc_sh, gsem = rest
        cid = lax.axis_index("c")
        sid = lax.axis_index("s")
        wid = sid * NC + cid
        base = sid * rpt

        zvec = jnp.zeros((LANES,), jnp.float32)
        for r in range(zrows):
            for j in range(d // LANES):
                zb[r, pl.ds(j * LANES, LANES)] = zvec
        for k in range(rpt // zrows):
            pltpu.sync_copy(zb, acc_sh.at[pl.ds(base + k * zrows, zrows)])
        if with_deg:
            for r in range(zrows):
                zbd[r, pl.ds(0, LANES)] = zvec
            for k in range(rpt // zrows):
                pltpu.sync_copy(zbd, deg_sh.at[pl.ds(base + k * zrows, zrows)])
            ovec = jnp.ones((LANES,), jnp.float32)
            for r in range(CH):
                ones[r, pl.ds(0, LANES)] = ovec
        @pl.when(sid == NS - 1)
        def _zero_tail():
            for k in range(tail // zrows):
                pltpu.sync_copy(zb, acc_sh.at[pl.ds(NS * rpt + k * zrows, zrows)])
                if with_deg:
                    pltpu.sync_copy(
                        zbd, deg_sh.at[pl.ds(NS * rpt + k * zrows, zrows)])
        plsc.subcore_barrier()

        nch_w = (nch - wid + nw - 1) // nw

        def step(i, carry):
            off = (wid + i * nw) * CH
            pltpu.sync_copy(src_h.at[pl.ds(off, CH)], sidx)
            pltpu.sync_copy(dst_h.at[pl.ds(off, CH)], didx)
            pltpu.async_copy(table_h.at[sidx], rows, gsem).wait()
            pltpu.sync_copy(rows, acc_sh.at[didx], add=True)
            if with_deg:
                pltpu.sync_copy(ones, deg_sh.at[didx], add=True)
            return carry

        lax.fori_loop(0, nch_w, step, 0)
        plsc.subcore_barrier()

        pltpu.sync_copy(acc_sh.at[pl.ds(base, rpt)],
                        acc_out.at[cid, pl.ds(base, rpt)])
        if with_deg:
            pltpu.sync_copy(deg_sh.at[pl.ds(base, rpt)],
                            deg_out.at[cid, pl.ds(base, rpt)])

        @pl.when(sid == NS - 1)
        def _copy_tail():
            pltpu.sync_copy(acc_sh.at[pl.ds(NS * rpt, tail)],
                            acc_out.at[cid, pl.ds(NS * rpt, tail)])
            if with_deg:
                pltpu.sync_copy(deg_sh.at[pl.ds(NS * rpt, tail)],
                                deg_out.at[cid, pl.ds(NS * rpt, tail)])

    fn = pl.kernel(body, out_type=out_type, mesh=mesh, scratch_types=scratch,
                   compiler_params=pltpu.CompilerParams(use_tc_tiling_on_sc=False))
    return fn(table, src, dst)


def _tc_transform(x, wa, wb):
    n, d = x.shape
    blk = 2000

    def body(x_ref, wa_ref, wb_ref, oa_ref, ob_ref):
        xv = x_ref[...]
        oa_ref[...] = jnp.dot(xv, wa_ref[...], preferred_element_type=jnp.float32)
        ob_ref[...] = jnp.dot(xv, wb_ref[...], preferred_element_type=jnp.float32)

    return pl.pallas_call(
        body,
        grid=(n // blk,),
        in_specs=[pl.BlockSpec((blk, d), lambda i: (i, 0)),
                  pl.BlockSpec((d, d), lambda i: (0, 0)),
                  pl.BlockSpec((d, d), lambda i: (0, 0))],
        out_specs=[pl.BlockSpec((blk, d), lambda i: (i, 0)),
                   pl.BlockSpec((blk, d), lambda i: (i, 0))],
        out_shape=[jax.ShapeDtypeStruct((n, d), jnp.float32)] * 2,
    )(x, wa, wb)


def _combine(s_ref, accp_ref, degp_ref, b_ref, relu):
    deg = degp_ref[0] + degp_ref[1]
    inv = 1.0 / jnp.maximum(deg[:, 0:1], 1.0)
    h = s_ref[...] + (accp_ref[0] + accp_ref[1]) * inv + b_ref[...][None, :]
    if relu:
        h = jnp.maximum(h, 0.0)
    return h


def _tc_combine_transform(s, accp, degp, b, wn, ws):
    _, n, d = accp.shape
    blk = 2000

    def body(s_ref, accp_ref, degp_ref, b_ref, wn_ref, ws_ref, on_ref, os_ref):
        h = _combine(s_ref, accp_ref, degp_ref, b_ref, relu=True)
        on_ref[...] = jnp.dot(h, wn_ref[...], preferred_element_type=jnp.float32)
        os_ref[...] = jnp.dot(h, ws_ref[...], preferred_element_type=jnp.float32)

    return pl.pallas_call(
        body,
        grid=(n // blk,),
        in_specs=[pl.BlockSpec((blk, d), lambda i: (i, 0)),
                  pl.BlockSpec((NC, blk, d), lambda i: (0, i, 0)),
                  pl.BlockSpec((NC, blk, DEGW), lambda i: (0, i, 0)),
                  pl.BlockSpec((d,), lambda i: (0,)),
                  pl.BlockSpec((d, d), lambda i: (0, 0)),
                  pl.BlockSpec((d, d), lambda i: (0, 0))],
        out_specs=[pl.BlockSpec((blk, d), lambda i: (i, 0)),
                   pl.BlockSpec((blk, d), lambda i: (i, 0))],
        out_shape=[jax.ShapeDtypeStruct((n, d), jnp.float32)] * 2,
    )(s, accp, degp, b, wn, ws)


def _tc_combine(s, accp, degp, b):
    _, n, d = accp.shape
    blk = 2000

    def body(s_ref, accp_ref, degp_ref, b_ref, o_ref):
        o_ref[...] = _combine(s_ref, accp_ref, degp_ref, b_ref, relu=False)

    return pl.pallas_call(
        body,
        grid=(n // blk,),
        in_specs=[pl.BlockSpec((blk, d), lambda i: (i, 0)),
                  pl.BlockSpec((NC, blk, d), lambda i: (0, i, 0)),
                  pl.BlockSpec((NC, blk, DEGW), lambda i: (0, i, 0)),
                  pl.BlockSpec((d,), lambda i: (0,))],
        out_specs=pl.BlockSpec((blk, d), lambda i: (i, 0)),
        out_shape=jax.ShapeDtypeStruct((n, d), jnp.float32),
    )(s, accp, degp, b)


def kernel(feat, edge_index, W_self0, W_neigh0, b0, W_self1, W_neigh1, b1):
    src = edge_index[0]
    dst = edge_index[1]
    t0, s0 = _tc_transform(feat, W_neigh0, W_self0)
    accp0, degp = _sc_aggregate(t0, src, dst, with_deg=True)
    t1, s1 = _tc_combine_transform(s0, accp0, degp, b0, W_neigh1, W_self1)
    (accp1,) = _sc_aggregate(t1, src, dst, with_deg=False)
    return _tc_combine(s1, accp1, degp, b1)


# speedup vs baseline: 12.8690x; 1.9397x over previous
import functools

import jax
import jax.numpy as jnp
from jax import lax
from jax.experimental import pallas as pl
from jax.experimental.pallas import tpu as pltpu
from jax.experimental.pallas import tpu_sc as plsc

NC = 2
NS = 16
LANES = 16
CH = 128
DEGW = 16
HALF = 40


def _sc_aggregate(table, src, dst):
    n, d = table.shape
    nw = NC * NS
    nch = src.shape[0] - HALF
    nch_lo = nch // nw
    nch_rem = nch % nw
    assert nch_lo + 1 <= 2 * HALF
    rpt = (n // NS) // 8 * 8
    tail = n - NS * rpt
    zrows = 8
    assert rpt % zrows == 0 and tail % zrows == 0

    mesh = plsc.VectorSubcoreMesh(
        core_axis_name="c", subcore_axis_name="s",
        num_cores=NC, num_subcores=NS)

    scratch = [
        pltpu.VMEM((HALF, CH), jnp.int32),
        pltpu.VMEM((HALF, CH), jnp.int32),
        pltpu.VMEM((CH, d), jnp.float32),
        pltpu.VMEM((CH, d), jnp.float32),
        pltpu.VMEM((zrows, d), jnp.float32),
        pltpu.VMEM_SHARED((n, d), jnp.float32),
        pltpu.SemaphoreType.DMA,
        pltpu.SemaphoreType.DMA,
    ]

    def body(table_h, src_h, dst_h, acc_out,
             sidx, didx, buf0, buf1, zb, acc_sh, sem0, sem1):
        cid = lax.axis_index("c")
        sid = lax.axis_index("s")
        wid = sid * NC + cid
        base = sid * rpt

        zvec = jnp.zeros((LANES,), jnp.float32)
        for r in range(zrows):
            for j in range(d // LANES):
                zb[r, pl.ds(j * LANES, LANES)] = zvec
        for k in range(rpt // zrows):
            pltpu.sync_copy(zb, acc_sh.at[pl.ds(base + k * zrows, zrows)])

        @pl.when(sid == NS - 1)
        def _zero_tail():
            for k in range(tail // zrows):
                pltpu.sync_copy(zb, acc_sh.at[pl.ds(NS * rpt + k * zrows, zrows)])
        plsc.subcore_barrier()

        start = wid * nch_lo + jnp.minimum(wid, nch_rem)
        n_i = nch_lo + jnp.where(wid < nch_rem, 1, 0)

        def start_gather(i, buf, sem):
            pltpu.async_copy(table_h.at[sidx.at[i]], buf, sem)

        def finish_chunk(i, buf, sem):
            pltpu.make_async_copy(table_h.at[sidx.at[i]], buf, sem).wait()
            pltpu.sync_copy(buf, acc_sh.at[didx.at[i]], add=True)

        for h in range(2):
            cnt = jnp.clip(n_i - h * HALF, 0, HALF)
            pltpu.sync_copy(src_h.at[pl.ds(start + h * HALF, HALF)], sidx)
            pltpu.sync_copy(dst_h.at[pl.ds(start + h * HALF, HALF)], didx)

            @pl.when(cnt > 0)
            def _prime():
                start_gather(0, buf0, sem0)

            def pair(p, carry):
                i0 = 2 * p

                @pl.when(i0 + 1 < cnt)
                def _g1():
                    start_gather(i0 + 1, buf1, sem1)

                finish_chunk(i0, buf0, sem0)

                @pl.when(i0 + 2 < cnt)
                def _g0():
                    start_gather(i0 + 2, buf0, sem0)

                @pl.when(i0 + 1 < cnt)
                def _f1():
                    finish_chunk(i0 + 1, buf1, sem1)

                return carry

            lax.fori_loop(0, (cnt + 1) // 2, pair, 0)
        plsc.subcore_barrier()

        pltpu.sync_copy(acc_sh.at[pl.ds(base, rpt)],
                        acc_out.at[cid, pl.ds(base, rpt)])

        @pl.when(sid == NS - 1)
        def _copy_tail():
            pltpu.sync_copy(acc_sh.at[pl.ds(NS * rpt, tail)],
                            acc_out.at[cid, pl.ds(NS * rpt, tail)])

    fn = pl.kernel(body,
                   out_type=jax.ShapeDtypeStruct((NC, n, d), jnp.float32),
                   mesh=mesh, scratch_types=scratch,
                   compiler_params=pltpu.CompilerParams(use_tc_tiling_on_sc=False))
    return fn(table, src, dst)


def _sc_degree(dst, n):
    nw = NC * NS
    nch = dst.shape[0] - HALF
    nch_lo = nch // nw
    nch_rem = nch % nw
    rpt = (n // NS) // 8 * 8
    tail = n - NS * rpt
    zrows = 8
    FIRE = 8

    mesh = plsc.VectorSubcoreMesh(
        core_axis_name="c", subcore_axis_name="s",
        num_cores=NC, num_subcores=NS)
    scratch = [
        pltpu.VMEM((nch_lo + 1, CH), jnp.int32),
        pltpu.VMEM((CH, DEGW), jnp.float32),
        pltpu.VMEM((zrows, DEGW), jnp.float32),
        pltpu.VMEM_SHARED((n, DEGW), jnp.float32),
        pltpu.SemaphoreType.DMA,
    ]

    def body(dst_h, deg_out, didx, ones, zbd, deg_sh, sem):
        cid = lax.axis_index("c")
        sid = lax.axis_index("s")
        wid = sid * NC + cid
        base = sid * rpt

        zvec = jnp.zeros((LANES,), jnp.float32)
        for r in range(zrows):
            zbd[r, pl.ds(0, LANES)] = zvec
        for k in range(rpt // zrows):
            pltpu.sync_copy(zbd, deg_sh.at[pl.ds(base + k * zrows, zrows)])

        @pl.when(sid == NS - 1)
        def _zero_tail():
            for k in range(tail // zrows):
                pltpu.sync_copy(zbd, deg_sh.at[pl.ds(NS * rpt + k * zrows, zrows)])
        ovec = jnp.ones((LANES,), jnp.float32)
        for r in range(CH):
            ones[r, pl.ds(0, LANES)] = ovec
        plsc.subcore_barrier()

        start = wid * nch_lo + jnp.minimum(wid, nch_rem)
        n_i = nch_lo + jnp.where(wid < nch_rem, 1, 0)
        pltpu.sync_copy(dst_h.at[pl.ds(start, nch_lo + 1)], didx)

        def group(g, carry):
            i0 = g * FIRE
            for j in range(FIRE):
                @pl.when(i0 + j < n_i)
                def _fire():
                    pltpu.async_copy(ones, deg_sh.at[didx.at[i0 + j]], sem,
                                     add=True)
            for j in range(FIRE):
                @pl.when(i0 + j < n_i)
                def _drain():
                    pltpu.make_async_copy(
                        ones, deg_sh.at[didx.at[i0 + j]], sem).wait()
            return carry

        lax.fori_loop(0, (n_i + FIRE - 1) // FIRE, group, 0)
        plsc.subcore_barrier()

        pltpu.sync_copy(deg_sh.at[pl.ds(base, rpt)],
                        deg_out.at[cid, pl.ds(base, rpt)])

        @pl.when(sid == NS - 1)
        def _copy_tail():
            pltpu.sync_copy(deg_sh.at[pl.ds(NS * rpt, tail)],
                            deg_out.at[cid, pl.ds(NS * rpt, tail)])

    fn = pl.kernel(body,
                   out_type=jax.ShapeDtypeStruct((NC, n, DEGW), jnp.float32),
                   mesh=mesh, scratch_types=scratch,
                   compiler_params=pltpu.CompilerParams(use_tc_tiling_on_sc=False))
    return fn(dst)


def _tc_transform(x, wa, wb):
    n, d = x.shape
    blk = 2000

    def body(x_ref, wa_ref, wb_ref, oa_ref, ob_ref):
        xv = x_ref[...]
        oa_ref[...] = jnp.dot(xv, wa_ref[...], preferred_element_type=jnp.float32)
        ob_ref[...] = jnp.dot(xv, wb_ref[...], preferred_element_type=jnp.float32)

    return pl.pallas_call(
        body,
        grid=(n // blk,),
        in_specs=[pl.BlockSpec((blk, d), lambda i: (i, 0)),
                  pl.BlockSpec((d, d), lambda i: (0, 0)),
                  pl.BlockSpec((d, d), lambda i: (0, 0))],
        out_specs=[pl.BlockSpec((blk, d), lambda i: (i, 0)),
                   pl.BlockSpec((blk, d), lambda i: (i, 0))],
        out_shape=[jax.ShapeDtypeStruct((n, d), jnp.float32)] * 2,
    )(x, wa, wb)


def _combine(s_ref, accp_ref, degp_ref, b_ref, relu):
    deg = degp_ref[0] + degp_ref[1]
    inv = 1.0 / jnp.maximum(deg[:, 0:1], 1.0)
    h = s_ref[...] + (accp_ref[0] + accp_ref[1]) * inv + b_ref[...][None, :]
    if relu:
        h = jnp.maximum(h, 0.0)
    return h


def _tc_combine_transform(s, accp, degp, b, wn, ws):
    _, n, d = accp.shape
    blk = 2000

    def body(s_ref, accp_ref, degp_ref, b_ref, wn_ref, ws_ref, on_ref, os_ref):
        h = _combine(s_ref, accp_ref, degp_ref, b_ref, relu=True)
        on_ref[...] = jnp.dot(h, wn_ref[...], preferred_element_type=jnp.float32)
        os_ref[...] = jnp.dot(h, ws_ref[...], preferred_element_type=jnp.float32)

    return pl.pallas_call(
        body,
        grid=(n // blk,),
        in_specs=[pl.BlockSpec((blk, d), lambda i: (i, 0)),
                  pl.BlockSpec((NC, blk, d), lambda i: (0, i, 0)),
                  pl.BlockSpec((NC, blk, DEGW), lambda i: (0, i, 0)),
                  pl.BlockSpec((d,), lambda i: (0,)),
                  pl.BlockSpec((d, d), lambda i: (0, 0)),
                  pl.BlockSpec((d, d), lambda i: (0, 0))],
        out_specs=[pl.BlockSpec((blk, d), lambda i: (i, 0)),
                   pl.BlockSpec((blk, d), lambda i: (i, 0))],
        out_shape=[jax.ShapeDtypeStruct((n, d), jnp.float32)] * 2,
    )(s, accp, degp, b, wn, ws)


def _tc_combine(s, accp, degp, b):
    _, n, d = accp.shape
    blk = 2000

    def body(s_ref, accp_ref, degp_ref, b_ref, o_ref):
        o_ref[...] = _combine(s_ref, accp_ref, degp_ref, b_ref, relu=False)

    return pl.pallas_call(
        body,
        grid=(n // blk,),
        in_specs=[pl.BlockSpec((blk, d), lambda i: (i, 0)),
                  pl.BlockSpec((NC, blk, d), lambda i: (0, i, 0)),
                  pl.BlockSpec((NC, blk, DEGW), lambda i: (0, i, 0)),
                  pl.BlockSpec((d,), lambda i: (0,))],
        out_specs=pl.BlockSpec((blk, d), lambda i: (i, 0)),
        out_shape=jax.ShapeDtypeStruct((n, d), jnp.float32),
    )(s, accp, degp, b)


def kernel(feat, edge_index, W_self0, W_neigh0, b0, W_self1, W_neigh1, b1):
    pad = jnp.zeros((HALF, CH), jnp.int32)
    src = jnp.concatenate([edge_index[0].reshape(-1, CH), pad])
    dst = jnp.concatenate([edge_index[1].reshape(-1, CH), pad])
    t0, s0 = _tc_transform(feat, W_neigh0, W_self0)
    accp0 = _sc_aggregate(t0, src, dst)
    degp = _sc_degree(dst, feat.shape[0])
    t1, s1 = _tc_combine_transform(s0, accp0, degp, b0, W_neigh1, W_self1)
    accp1 = _sc_aggregate(t1, src, dst)
    return _tc_combine(s1, accp1, degp, b1)
